# baseline (speedup 1.0000x reference)
import functools

import jax
import jax.numpy as jnp
from jax import lax
from jax.experimental import pallas as pl
from jax.experimental.pallas import tpu as pltpu
from jax.experimental.pallas import tpu_sc as plsc

DMIN, DMAX, STEP = 0.0, 8.0, 0.25
VAR = STEP
NFILT = 33
NC, NS = 2, 16
SUBS_PER_CASE = 8


def _sc_dist2(coords_p, nbr_p, lat_pad, stack, B, N, M, NP):
    NT = NP // 128
    NTW = NT // NS
    LW = NTW * 128
    MT = M // 8
    CASE_STRIDE = MT * NT * 1024
    MT_STRIDE = NT * 1024
    mesh = plsc.VectorSubcoreMesh(
        core_axis_name="c", subcore_axis_name="s", num_cores=NC, num_subcores=NS
    )

    @functools.partial(
        pl.kernel,
        mesh=mesh,
        compiler_params=pltpu.CompilerParams(needs_layout_passes=False),
        out_type=jax.ShapeDtypeStruct((B * CASE_STRIDE,), jnp.float32),
        scratch_types=[
            pltpu.VMEM((NP,), jnp.float32),
            pltpu.VMEM((NP,), jnp.float32),
            pltpu.VMEM((NP,), jnp.float32),
            pltpu.VMEM((M * LW,), jnp.int32),
            pltpu.VMEM((MT * NTW * 1024,), jnp.float32),
            pltpu.VMEM((48,), jnp.float32),
            pltpu.SemaphoreType.DMA,
        ],
    )
    def k(coords_hbm, nbr_hbm, lat_hbm, out, xv, yv, zv, nb, ov, latv, sem):
        b = lax.axis_index("c")
        sub = lax.axis_index("s")
        nt0 = sub * NTW
        case = stack * B + b

        copies = []
        cbase = (b * 3) * 2 + stack
        copies.append(pltpu.make_async_copy(
            coords_hbm.at[pl.ds((cbase + 0) * NP, NP)], xv, sem))
        copies.append(pltpu.make_async_copy(
            coords_hbm.at[pl.ds((cbase + 2) * NP, NP)], yv, sem))
        copies.append(pltpu.make_async_copy(
            coords_hbm.at[pl.ds((cbase + 4) * NP, NP)], zv, sem))
        for m in range(M):
            off = ((b * M + m) * 2 + stack) * NP + nt0 * 128
            copies.append(pltpu.make_async_copy(
                nbr_hbm.at[pl.ds(off, LW)], nb.at[pl.ds(m * LW, LW)], sem))
        for cp in copies:
            cp.start()
        pltpu.sync_copy(lat_hbm.at[pl.ds(case * 48, 48)], latv)
        lxv = latv[pl.ds(0, 16)]
        lyv = latv[pl.ds(16, 16)]
        lzv = latv[pl.ds(32, 16)]
        for cp in copies:
            cp.wait()

        def pbody(p, _):
            base = p * 16
            ax = xv[pl.ds(nt0 * 128 + base, 16)]
            ay = yv[pl.ds(nt0 * 128 + base, 16)]
            az = zv[pl.ds(nt0 * 128 + base, 16)]
            t = base // 128
            j = base - t * 128
            odst = t * 1024 + j

            @plsc.parallel_loop(0, M, unroll=4)
            def mbody(m):
                idx = nb[pl.ds(m * LW + base, 16)]
                adx = jnp.abs(ax - plsc.load_gather(xv, [idx]))
                ady = jnp.abs(ay - plsc.load_gather(yv, [idx]))
                adz = jnp.abs(az - plsc.load_gather(zv, [idx]))
                ex = jnp.minimum(adx, lxv - adx)
                ey = jnp.minimum(ady, lyv - ady)
                ez = jnp.minimum(adz, lzv - adz)
                ov[pl.ds((m // 8) * (NTW * 1024) + (m % 8) * 128 + odst, 16)] = (
                    ex * ex + ey * ey + ez * ez
                )

            return 0

        lax.fori_loop(0, LW // 16, pbody, 0)

        for mt in range(MT):
            pltpu.sync_copy(
                ov.at[pl.ds(mt * NTW * 1024, NTW * 1024)],
                out.at[pl.ds(b * CASE_STRIDE + mt * MT_STRIDE + nt0 * 1024,
                             NTW * 1024)],
            )

    return k(coords_p, nbr_p, lat_pad)


def _tc_expand(d2v, B, N, M, NP):
    MT = M // 8
    NT = NP // 128
    NTB = 5 if NT % 5 == 0 else (8 if NT % 8 == 0 else 1)
    NL = NTB * 128
    nblk = NT // NTB
    inv_var2 = 1.0 / (VAR * VAR)

    def body(d2_ref, o_ref):
        d = jnp.sqrt(d2_ref[...])
        for mt in range(MT):
            for t in range(NTB):
                dv = d[0, mt, t]
                for k in range(NFILT):
                    diff = dv - (k * STEP)
                    o_ref[0, k, mt * 8:(mt + 1) * 8, t * 128:(t + 1) * 128] = (
                        jnp.exp(diff * diff * (-inv_var2))
                    )

    return pl.pallas_call(
        body,
        grid=(B, nblk),
        in_specs=[
            pl.BlockSpec((1, MT, NTB, 8, 128),
                         lambda b, t: (b, 0, t, 0, 0)),
        ],
        out_specs=pl.BlockSpec((1, NFILT, M, NL), lambda b, t: (b, 0, 0, t)),
        out_shape=jax.ShapeDtypeStruct((B, NFILT, M, N), jnp.float32),
    )(d2v)


def kernel(stacked_coords, stacked_lattices, stacked_nbr_lists):
    B, N = stacked_coords.shape[0], stacked_coords.shape[1]
    M = stacked_nbr_lists.shape[2]
    NP = -(-N // (SUBS_PER_CASE * 128)) * (SUBS_PER_CASE * 128)

    nbr1 = stacked_nbr_lists[..., 0]
    nbr2 = stacked_nbr_lists[..., 1]

    coords_p = jnp.pad(
        jnp.transpose(stacked_coords, (0, 2, 3, 1)),
        ((0, 0), (0, 0), (0, 0), (0, NP - N)),
    ).reshape(-1)
    nbr_p = jnp.pad(
        jnp.transpose(stacked_nbr_lists, (0, 2, 3, 1)),
        ((0, 0), (0, 0), (0, 0), (0, NP - N)),
    ).reshape(-1)
    lat_t = jnp.transpose(stacked_lattices, (2, 0, 1))
    lat_pad = jnp.broadcast_to(lat_t[..., None], (2, B, 3, 16)).reshape(-1)

    d2f1 = _sc_dist2(coords_p, nbr_p, lat_pad, 0, B, N, M, NP)
    d2f2 = _sc_dist2(coords_p, nbr_p, lat_pad, 1, B, N, M, NP)
    shape5 = (B, M // 8, NP // 128, 8, 128)
    bf1 = _tc_expand(d2f1.reshape(shape5), B, N, M, NP)
    bf2 = _tc_expand(d2f2.reshape(shape5), B, N, M, NP)
    bond_fea_1 = jnp.transpose(bf1, (0, 3, 2, 1))
    bond_fea_2 = jnp.transpose(bf2, (0, 3, 2, 1))
    return (nbr1, bond_fea_1, nbr2, bond_fea_2)


# speedup vs baseline: 1.1473x; 1.1473x over previous
import functools

import jax
import jax.numpy as jnp
from jax import lax
from jax.experimental import pallas as pl
from jax.experimental.pallas import tpu as pltpu
from jax.experimental.pallas import tpu_sc as plsc

DMIN, DMAX, STEP = 0.0, 8.0, 0.25
VAR = STEP
NFILT = 33
NC, NS = 2, 16
SUBS_PER_CASE = 8


def _sc_dist2(coords_p, nbr_p, lat_pad, stack, B, N, M, NP):
    NT = NP // 128
    NTW = NT // NS
    LW = NTW * 128
    MT = M // 8
    CASE_STRIDE = MT * NT * 1024
    MT_STRIDE = NT * 1024
    mesh = plsc.VectorSubcoreMesh(
        core_axis_name="c", subcore_axis_name="s", num_cores=NC, num_subcores=NS
    )

    @functools.partial(
        pl.kernel,
        mesh=mesh,
        compiler_params=pltpu.CompilerParams(needs_layout_passes=False),
        out_type=jax.ShapeDtypeStruct((B * CASE_STRIDE,), jnp.float32),
        scratch_types=[
            pltpu.VMEM((NP,), jnp.float32),
            pltpu.VMEM((NP,), jnp.float32),
            pltpu.VMEM((NP,), jnp.float32),
            pltpu.VMEM((M * LW,), jnp.int32),
            pltpu.VMEM((MT * NTW * 1024,), jnp.float32),
            pltpu.VMEM((48,), jnp.float32),
            pltpu.SemaphoreType.DMA,
        ],
    )
    def k(coords_hbm, nbr_hbm, lat_hbm, out, xv, yv, zv, nb, ov, latv, sem):
        b = lax.axis_index("c")
        sub = lax.axis_index("s")
        nt0 = sub * NTW
        case = stack * B + b

        copies = []
        cbase = (b * 3) * 2 + stack
        copies.append(pltpu.make_async_copy(
            coords_hbm.at[pl.ds((cbase + 0) * NP, NP)], xv, sem))
        copies.append(pltpu.make_async_copy(
            coords_hbm.at[pl.ds((cbase + 2) * NP, NP)], yv, sem))
        copies.append(pltpu.make_async_copy(
            coords_hbm.at[pl.ds((cbase + 4) * NP, NP)], zv, sem))
        for m in range(M):
            off = ((b * M + m) * 2 + stack) * NP + nt0 * 128
            copies.append(pltpu.make_async_copy(
                nbr_hbm.at[pl.ds(off, LW)], nb.at[pl.ds(m * LW, LW)], sem))
        for cp in copies:
            cp.start()
        pltpu.sync_copy(lat_hbm.at[pl.ds(case * 48, 48)], latv)
        lxv = latv[pl.ds(0, 16)]
        lyv = latv[pl.ds(16, 16)]
        lzv = latv[pl.ds(32, 16)]
        for cp in copies:
            cp.wait()

        def pbody(p, _):
            base = p * 16
            ax = xv[pl.ds(nt0 * 128 + base, 16)]
            ay = yv[pl.ds(nt0 * 128 + base, 16)]
            az = zv[pl.ds(nt0 * 128 + base, 16)]
            t = base // 128
            j = base - t * 128
            odst = t * 1024 + j

            @plsc.parallel_loop(0, M, unroll=4)
            def mbody(m):
                idx = nb[pl.ds(m * LW + base, 16)]
                adx = jnp.abs(ax - plsc.load_gather(xv, [idx]))
                ady = jnp.abs(ay - plsc.load_gather(yv, [idx]))
                adz = jnp.abs(az - plsc.load_gather(zv, [idx]))
                ex = jnp.minimum(adx, lxv - adx)
                ey = jnp.minimum(ady, lyv - ady)
                ez = jnp.minimum(adz, lzv - adz)
                ov[pl.ds((m // 8) * (NTW * 1024) + (m % 8) * 128 + odst, 16)] = (
                    ex * ex + ey * ey + ez * ez
                )

            return 0

        lax.fori_loop(0, LW // 16, pbody, 0)

        for mt in range(MT):
            pltpu.sync_copy(
                ov.at[pl.ds(mt * NTW * 1024, NTW * 1024)],
                out.at[pl.ds(b * CASE_STRIDE + mt * MT_STRIDE + nt0 * 1024,
                             NTW * 1024)],
            )

    return k(coords_p, nbr_p, lat_pad)


def _tc_expand(d2v, B, N, M, NP):
    MT = M // 8
    NT = NP // 128
    NTB = 20 if NT % 20 == 0 else (8 if NT % 8 == 0 else 1)
    NL = NTB * 128
    nblk = NT // NTB
    inv_var2 = 1.0 / (VAR * VAR)

    def body(d2_ref, o_ref):
        d = jnp.sqrt(d2_ref[...])
        for mt in range(MT):
            for t in range(NTB):
                dv = d[0, mt, t]
                for k in range(NFILT):
                    diff = dv - (k * STEP)
                    o_ref[0, k, mt * 8:(mt + 1) * 8, t * 128:(t + 1) * 128] = (
                        jnp.exp(diff * diff * (-inv_var2))
                    )

    return pl.pallas_call(
        body,
        grid=(B, nblk),
        in_specs=[
            pl.BlockSpec((1, MT, NTB, 8, 128),
                         lambda b, t: (b, 0, t, 0, 0)),
        ],
        out_specs=pl.BlockSpec((1, NFILT, M, NL), lambda b, t: (b, 0, 0, t)),
        out_shape=jax.ShapeDtypeStruct((B, NFILT, M, N), jnp.float32),
    )(d2v)


def kernel(stacked_coords, stacked_lattices, stacked_nbr_lists):
    B, N = stacked_coords.shape[0], stacked_coords.shape[1]
    M = stacked_nbr_lists.shape[2]
    NP = -(-N // (SUBS_PER_CASE * 128)) * (SUBS_PER_CASE * 128)

    nbr1 = stacked_nbr_lists[..., 0]
    nbr2 = stacked_nbr_lists[..., 1]

    coords_p = jnp.pad(
        jnp.transpose(stacked_coords, (0, 2, 3, 1)),
        ((0, 0), (0, 0), (0, 0), (0, NP - N)),
    ).reshape(-1)
    nbr_p = jnp.pad(
        jnp.transpose(stacked_nbr_lists, (0, 2, 3, 1)),
        ((0, 0), (0, 0), (0, 0), (0, NP - N)),
    ).reshape(-1)
    lat_t = jnp.transpose(stacked_lattices, (2, 0, 1))
    lat_pad = jnp.broadcast_to(lat_t[..., None], (2, B, 3, 16)).reshape(-1)

    d2f1 = _sc_dist2(coords_p, nbr_p, lat_pad, 0, B, N, M, NP)
    d2f2 = _sc_dist2(coords_p, nbr_p, lat_pad, 1, B, N, M, NP)
    shape5 = (B, M // 8, NP // 128, 8, 128)
    bf1 = _tc_expand(d2f1.reshape(shape5), B, N, M, NP)
    bf2 = _tc_expand(d2f2.reshape(shape5), B, N, M, NP)
    bond_fea_1 = jnp.transpose(bf1, (0, 3, 2, 1))
    bond_fea_2 = jnp.transpose(bf2, (0, 3, 2, 1))
    return (nbr1, bond_fea_1, nbr2, bond_fea_2)
